# baseline (speedup 1.0000x reference)
import jax
import jax.numpy as jnp
from jax import lax
from jax.experimental import pallas as pl
from jax.experimental.pallas import tpu as pltpu
from jax.experimental.pallas import tpu_sc as plsc

NUM_ENT = 100000
NUM_REL = 1000
NUM_TIME = 1000
EMB = 128
T_EMB = 64
ALP = 0.5
B = 16384
H = 50
HR = 64

NC = 2
NS = 16
NW = NC * NS
BPW = B // NW
CB = 64
NCHUNK = BPW // CB
NBLK = NW * NCHUNK
NV = EMB // 16
PKW = EMB // 2
NPW = PKW // 16

_GATHER_DNUMS = lax.GatherDimensionNumbers(
    offset_dims=(), collapsed_slice_dims=(0,), start_index_map=(0,))


def _lane_gather(x, idx):
    return lax.gather(
        x, idx[:, None], _GATHER_DNUMS, slice_sizes=(1,),
        mode=lax.GatherScatterMode.PROMISE_IN_BOUNDS)


def _unpack(v16):
    bits = lax.bitcast_convert_type(v16, jnp.int32)
    lo = lax.bitcast_convert_type(
        lax.shift_left(bits, 16), jnp.float32)
    hi = lax.bitcast_convert_type(
        lax.bitwise_and(bits, jnp.int32(-65536)), jnp.float32)
    return lo, hi


def _rel_read(rel_tile, idx, v):
    return rel_tile[idx >> 1, pl.ds((idx & 1) * PKW + v * 16, 16)]


def _sc_body(ehiss_hbm, rhiss_hbm, heads_hbm, rels_hbm, tails_hbm,
             dateid_hbm, ent_p2_hbm, rel_p_hbm, tim_w, out_hbm,
             ehiss_v, rhiss_v, heads_v, rels_v, tails_v, dateid_v,
             h_rows, t_rows, r_rows, t1_rows, g_ent, rel_tile,
             ssq_v, scores_v, sems):
    wid = lax.axis_index("s") * NC + lax.axis_index("c")
    ent_p_hbm = ent_p2_hbm

    pltpu.sync_copy(rel_p_hbm, rel_tile)

    def issue_elem(j, pb):
        pltpu.async_copy(
            ent_p_hbm.at[ehiss_v.at[j, pl.ds(0, 24)]],
            g_ent.at[pb, pl.ds(0, 24)], sems.at[pb])
        pltpu.async_copy(
            ent_p_hbm.at[ehiss_v.at[j, pl.ds(24, 26)]],
            g_ent.at[pb, pl.ds(24, 26)], sems.at[pb])

    def wait_elem(j, pb):
        pltpu.make_async_copy(
            ent_p_hbm.at[ehiss_v.at[j, pl.ds(0, 24)]],
            g_ent.at[pb, pl.ds(0, 24)], sems.at[pb]).wait()
        pltpu.make_async_copy(
            ent_p_hbm.at[ehiss_v.at[j, pl.ds(24, 26)]],
            g_ent.at[pb, pl.ds(24, 26)], sems.at[pb]).wait()

    def chunk_body(c, _):
        blk = wid * NCHUNK + c
        pltpu.sync_copy(ehiss_hbm.at[pl.ds(blk * CB, CB)], ehiss_v)
        pltpu.sync_copy(rhiss_hbm.at[pl.ds(blk * CB, CB)], rhiss_v)
        pltpu.sync_copy(heads_hbm.at[pl.ds(blk * CB, CB)], heads_v)
        pltpu.sync_copy(rels_hbm.at[pl.ds(blk * CB, CB)], rels_v)
        pltpu.sync_copy(tails_hbm.at[pl.ds(blk * CB, CB)], tails_v)
        pltpu.sync_copy(dateid_hbm.at[pl.ds(blk * CB, CB)], dateid_v)
        cp1 = pltpu.async_copy(ent_p_hbm.at[heads_v], h_rows, sems.at[0])
        cp2 = pltpu.async_copy(ent_p_hbm.at[tails_v], t_rows, sems.at[0])
        cp3 = pltpu.async_copy(tim_w.at[dateid_v], t1_rows, sems.at[0])
        cp1.wait()
        cp2.wait()
        cp3.wait()

        for g in range(CB // 16):
            rvec = rels_v[pl.ds(g * 16, 16)]
            for l in range(16):
                ridx = rvec[l]
                for v in range(NPW):
                    lo, hi = _unpack(_rel_read(rel_tile, ridx, v))
                    r_rows[g * 16 + l, pl.ds(v * 16, 16)] = lo
                    r_rows[g * 16 + l, pl.ds((v + NPW) * 16, 16)] = hi

        issue_elem(0, 0)
        issue_elem(1, 1)
        issue_elem(2, 2)

        def outer_body(j0, _):
            def elem_body(j1, ssq_vec):
                j = j0 * 16 + j1
                p = j & 3
                @pl.when(j < CB - 3)
                def _():
                    issue_elem(j + 3, (j + 3) & 3)

                zero = jnp.zeros((16,), jnp.float32)

                def relsum(rvec, nlanes, q):
                    for l in range(nlanes):
                        ridx = rvec[l]
                        for v in range(NPW):
                            lo, hi = _unpack(_rel_read(rel_tile, ridx, v))
                            q[v] = q[v] + lo
                            q[v + NPW] = q[v + NPW] + hi
                    return q

                def rel_body(g, q):
                    rvec = rhiss_v[j, pl.ds(g * 16, 16)]
                    return tuple(relsum(rvec, 16, list(q)))

                accsR = lax.fori_loop(0, 3, rel_body, (zero,) * NV)
                accsR = relsum(rhiss_v[j, pl.ds(48, 16)], H - 48,
                               list(accsR))

                wait_elem(j, p)

                def red_body(i, accs):
                    i2 = i * 2
                    new = list(accs)
                    for i3 in (i2, i2 + 1):
                        for v in range(NPW):
                            lo, hi = _unpack(g_ent[p, i3, pl.ds(v * 16, 16)])
                            new[v] = new[v] + lo
                            new[v + NPW] = new[v + NPW] + hi
                    return tuple(new)

                accsE = lax.fori_loop(0, H // 2, red_body, (zero,) * NV)

                sE = ALP / H
                acc16 = jnp.zeros((16,), jnp.float32)
                for v in range(NPW):
                    hlo, hhi = _unpack(h_rows[j, pl.ds(v * 16, 16)])
                    tlo, thi = _unpack(t_rows[j, pl.ds(v * 16, 16)])
                    for k, hv, tv in ((v, hlo, tlo), (v + NPW, hhi, thi)):
                        pv = sE * accsE[k]
                        qv = sE * accsR[k]
                        hh = (1.0 - ALP) * hv + pv
                        tt = (1.0 - ALP) * tv + pv
                        rr = ((1.0 - ALP) * r_rows[j, pl.ds(k * 16, 16)]
                              + qv)
                        prod = hh * rr * tt * t1_rows[j, pl.ds(k * 16, 16)]
                        acc16 = acc16 + prod * prod

                lane = lax.iota(jnp.int32, 16)
                for d in (1, 2, 4, 8):
                    acc16 = acc16 + _lane_gather(acc16, lane ^ d)
                return jnp.where(lane == j1, acc16, ssq_vec)

            ssq_vec = lax.fori_loop(
                0, 16, elem_body, jnp.zeros((16,), jnp.float32))
            ssq_v[pl.ds(j0 * 16, 16)] = ssq_vec
            return 0

        lax.fori_loop(0, CB // 16, outer_body, 0)

        for v in range(CB // 16):
            x = ssq_v[pl.ds(v * 16, 16)]
            bits = lax.bitcast_convert_type(x, jnp.int32)
            y = lax.bitcast_convert_type(
                lax.shift_right_logical(bits, 1) + 0x1FBD1DF6, jnp.float32)
            for _ in range(3):
                y = 0.5 * (y + x / y)
            scores_v[pl.ds(v * 16, 16)] = -y

        pltpu.sync_copy(scores_v, out_hbm.at[pl.ds(blk * CB, CB)])
        return 0

    lax.fori_loop(0, NCHUNK, chunk_body, 0)


def _pack_tbl(tbl):
    b = tbl.astype(jnp.bfloat16)
    u = lax.bitcast_convert_type(b, jnp.uint16).astype(jnp.uint32)
    packed = u[:, :PKW] | (u[:, PKW:] << 16)
    return lax.bitcast_convert_type(packed, jnp.float32)


def _pack_ent(tbl):
    bits = lax.bitcast_convert_type(tbl, jnp.int32)
    rne = bits + 0x7FFF + (lax.shift_right_logical(bits, 16) & 1)
    lo = lax.shift_right_logical(rne[:, :PKW], 16)
    hi = rne[:, PKW:] & jnp.int32(-65536)
    return lax.bitcast_convert_type(lo | hi, jnp.float32)


@jax.jit
def kernel(heads, rels, tails, dateid, hiss, ent_hiss, ent_w, rel_w, tim_w):
    mesh = plsc.VectorSubcoreMesh(
        core_axis_name="c", subcore_axis_name="s",
        num_cores=NC, num_subcores=NS)
    run = pl.kernel(
        _sc_body,
        out_type=jax.ShapeDtypeStruct((B,), jnp.float32),
        mesh=mesh,
        compiler_params=pltpu.CompilerParams(use_tc_tiling_on_sc=False),
        scratch_types=[
            pltpu.VMEM((CB, H), jnp.int32),
            pltpu.VMEM((CB, HR), jnp.int32),
            pltpu.VMEM((CB,), jnp.int32),
            pltpu.VMEM((CB,), jnp.int32),
            pltpu.VMEM((CB,), jnp.int32),
            pltpu.VMEM((CB,), jnp.int32),
            pltpu.VMEM((CB, PKW), jnp.float32),
            pltpu.VMEM((CB, PKW), jnp.float32),
            pltpu.VMEM((CB, EMB), jnp.float32),
            pltpu.VMEM((CB, EMB), jnp.float32),
            pltpu.VMEM((4, H, PKW), jnp.float32),
            pltpu.VMEM((NUM_REL // 2, EMB), jnp.float32),
            pltpu.VMEM((CB,), jnp.float32),
            pltpu.VMEM((CB,), jnp.float32),
            pltpu.SemaphoreType.DMA((4,)),
        ],
    )
    ent_p = _pack_ent(ent_w)
    rel_p = _pack_tbl(rel_w).reshape(NUM_REL // 2, EMB)
    tim_full = jnp.concatenate(
        [tim_w, jnp.ones((NUM_TIME, EMB - T_EMB), jnp.float32)], axis=1)
    rhiss_p = jnp.pad(hiss, ((0, 0), (0, HR - H)))
    return run(ent_hiss, rhiss_p, heads, rels, tails, dateid,
               ent_p, rel_p, tim_full)


# speedup vs baseline: 1.1343x; 1.1343x over previous
import jax
import jax.numpy as jnp
from jax import lax
from jax.experimental import pallas as pl
from jax.experimental.pallas import tpu as pltpu
from jax.experimental.pallas import tpu_sc as plsc

NUM_ENT = 100000
NUM_REL = 1000
NUM_TIME = 1000
EMB = 128
T_EMB = 64
ALP = 0.5
B = 16384
H = 50

NC = 2
NS = 16
NW = NC * NS
BPW = B // NW
CB = 64
NCHUNK = BPW // CB
NBLK = NW * NCHUNK
NV = EMB // 16
PKW = EMB // 2

_GATHER_DNUMS = lax.GatherDimensionNumbers(
    offset_dims=(), collapsed_slice_dims=(0,), start_index_map=(0,))


def _lane_gather(x, idx):
    return lax.gather(
        x, idx[:, None], _GATHER_DNUMS, slice_sizes=(1,),
        mode=lax.GatherScatterMode.PROMISE_IN_BOUNDS)


def _unpack(v16):
    bits = lax.bitcast_convert_type(v16, jnp.int32)
    lo = lax.bitcast_convert_type(
        lax.shift_left(bits, 16), jnp.float32)
    hi = lax.bitcast_convert_type(
        lax.bitwise_and(bits, jnp.int32(-65536)), jnp.float32)
    return lo, hi


def _rel_read(rel_tile, idx, v):
    return rel_tile[idx >> 1, pl.ds((idx & 1) * PKW + v * 16, 16)]


def _sc_body(ehiss_hbm, rhiss_hbm, heads_hbm, rels_hbm, tails_hbm,
             dateid_hbm, ent_w, rel_p_hbm, tim_w, out_hbm,
             ehiss_v, rhiss_v, heads_v, rels_v, tails_v, dateid_v,
             h_rows, t_rows, r_rows, t1_rows, g_ent, rel_tile,
             ssq_v, scores_v, sems):
    wid = lax.axis_index("s") * NC + lax.axis_index("c")

    pltpu.sync_copy(rel_p_hbm, rel_tile)

    def issue_elem(j, pb):
        pltpu.async_copy(ent_w.at[ehiss_v.at[j]], g_ent.at[pb], sems.at[pb])

    def wait_elem(j, pb):
        pltpu.make_async_copy(
            ent_w.at[ehiss_v.at[j]], g_ent.at[pb], sems.at[pb]).wait()

    def chunk_body(c, _):
        blk = wid * NCHUNK + c
        pltpu.sync_copy(ehiss_hbm.at[blk], ehiss_v)
        pltpu.sync_copy(rhiss_hbm.at[blk], rhiss_v)
        pltpu.sync_copy(heads_hbm.at[blk], heads_v)
        pltpu.sync_copy(rels_hbm.at[blk], rels_v)
        pltpu.sync_copy(tails_hbm.at[blk], tails_v)
        pltpu.sync_copy(dateid_hbm.at[blk], dateid_v)
        cp1 = pltpu.async_copy(ent_w.at[heads_v], h_rows, sems.at[0])
        cp2 = pltpu.async_copy(ent_w.at[tails_v], t_rows, sems.at[0])
        cp3 = pltpu.async_copy(tim_w.at[dateid_v], t1_rows, sems.at[0])
        cp1.wait()
        cp2.wait()
        cp3.wait()

        for g in range(CB // 16):
            rvec = rels_v[pl.ds(g * 16, 16)]
            for l in range(16):
                ridx = rvec[l]
                for v in range(PKW // 16):
                    lo, hi = _unpack(_rel_read(rel_tile, ridx, v))
                    r_rows[g * 16 + l, pl.ds((2 * v) * 16, 16)] = lo
                    r_rows[g * 16 + l, pl.ds((2 * v + 1) * 16, 16)] = hi

        issue_elem(0, 0)

        def outer_body(j0, _):
            def elem_body(j1, ssq_vec):
                j = j0 * 16 + j1
                p = j & 1
                @pl.when(j < CB - 1)
                def _():
                    issue_elem(j + 1, 1 - p)

                zero = jnp.zeros((16,), jnp.float32)

                def relsum(rvec, nlanes, q):
                    for l in range(nlanes):
                        ridx = rvec[l]
                        for v in range(PKW // 16):
                            lo, hi = _unpack(_rel_read(rel_tile, ridx, v))
                            q[2 * v] = q[2 * v] + lo
                            q[2 * v + 1] = q[2 * v + 1] + hi
                    return q

                def rel_body(g, q):
                    rvec = rhiss_v[j, pl.ds(g * 16, 16)]
                    return tuple(relsum(rvec, 16, list(q)))

                accsR = lax.fori_loop(0, 3, rel_body, (zero,) * NV)
                accsR = relsum(rhiss_v[j, pl.ds(48, 16)], H - 48,
                               list(accsR))

                wait_elem(j, p)

                def red_body(i, accs):
                    i4 = i * 4
                    return tuple(
                        accs[v]
                        + ((g_ent[p, i4, pl.ds(v * 16, 16)]
                            + g_ent[p, i4 + 1, pl.ds(v * 16, 16)])
                           + (g_ent[p, i4 + 2, pl.ds(v * 16, 16)]
                              + g_ent[p, i4 + 3, pl.ds(v * 16, 16)]))
                        for v in range(NV)
                    )

                accsE = lax.fori_loop(0, H // 4, red_body, (zero,) * NV)
                accsE = tuple(
                    accsE[v] + (g_ent[p, 48, pl.ds(v * 16, 16)]
                                + g_ent[p, 49, pl.ds(v * 16, 16)])
                    for v in range(NV))

                sE = ALP / H
                acc16 = jnp.zeros((16,), jnp.float32)
                for k in range(NV):
                    pv = sE * accsE[k]
                    qv = sE * accsR[k]
                    hv = (1.0 - ALP) * h_rows[j, pl.ds(k * 16, 16)] + pv
                    tv = (1.0 - ALP) * t_rows[j, pl.ds(k * 16, 16)] + pv
                    rr = (1.0 - ALP) * r_rows[j, pl.ds(k * 16, 16)] + qv
                    prod = hv * rr * tv * t1_rows[j, pl.ds(k * 16, 16)]
                    acc16 = acc16 + prod * prod

                lane = lax.iota(jnp.int32, 16)
                for d in (1, 2, 4, 8):
                    acc16 = acc16 + _lane_gather(acc16, lane ^ d)
                return jnp.where(lane == j1, acc16, ssq_vec)

            ssq_vec = lax.fori_loop(
                0, 16, elem_body, jnp.zeros((16,), jnp.float32))
            ssq_v[pl.ds(j0 * 16, 16)] = ssq_vec
            return 0

        lax.fori_loop(0, CB // 16, outer_body, 0)

        for v in range(CB // 16):
            x = ssq_v[pl.ds(v * 16, 16)]
            bits = lax.bitcast_convert_type(x, jnp.int32)
            y = lax.bitcast_convert_type(
                lax.shift_right_logical(bits, 1) + 0x1FBD1DF6, jnp.float32)
            for _ in range(3):
                y = 0.5 * (y + x / y)
            scores_v[pl.ds(v * 16, 16)] = -y

        pltpu.sync_copy(scores_v, out_hbm.at[pl.ds(blk * CB, CB)])
        return 0

    lax.fori_loop(0, NCHUNK, chunk_body, 0)


def _pack_rel(tbl):
    b = tbl.astype(jnp.bfloat16)
    u = lax.bitcast_convert_type(b, jnp.uint16).astype(jnp.uint32)
    u4 = u.reshape(tbl.shape[0], EMB // 32, 2, 16)
    packed = u4[:, :, 0, :] | (u4[:, :, 1, :] << 16)
    packed = lax.bitcast_convert_type(packed, jnp.float32)
    return packed.reshape(tbl.shape[0] // 2, EMB)


@jax.jit
def kernel(heads, rels, tails, dateid, hiss, ent_hiss, ent_w, rel_w, tim_w):
    mesh = plsc.VectorSubcoreMesh(
        core_axis_name="c", subcore_axis_name="s",
        num_cores=NC, num_subcores=NS)
    run = pl.kernel(
        _sc_body,
        out_type=jax.ShapeDtypeStruct((B,), jnp.float32),
        mesh=mesh,
        scratch_types=[
            pltpu.VMEM((CB, H), jnp.int32),
            pltpu.VMEM((CB, 64), jnp.int32),
            pltpu.VMEM((CB,), jnp.int32),
            pltpu.VMEM((CB,), jnp.int32),
            pltpu.VMEM((CB,), jnp.int32),
            pltpu.VMEM((CB,), jnp.int32),
            pltpu.VMEM((CB, EMB), jnp.float32),
            pltpu.VMEM((CB, EMB), jnp.float32),
            pltpu.VMEM((CB, EMB), jnp.float32),
            pltpu.VMEM((CB, EMB), jnp.float32),
            pltpu.VMEM((2, H, EMB), jnp.float32),
            pltpu.VMEM((NUM_REL // 2, EMB), jnp.float32),
            pltpu.VMEM((CB,), jnp.float32),
            pltpu.VMEM((CB,), jnp.float32),
            pltpu.SemaphoreType.DMA((2,)),
        ],
    )
    rel_p = _pack_rel(rel_w)
    tim_full = jnp.concatenate(
        [tim_w, jnp.ones((NUM_TIME, EMB - T_EMB), jnp.float32)], axis=1)
    ehiss_r = ent_hiss.reshape(NBLK, CB, H)
    rhiss_r = jnp.pad(hiss, ((0, 0), (0, 64 - H))).reshape(NBLK, CB, 64)
    heads_r = heads.reshape(NBLK, CB)
    rels_r = rels.reshape(NBLK, CB)
    tails_r = tails.reshape(NBLK, CB)
    dateid_r = dateid.reshape(NBLK, CB)
    return run(ehiss_r, rhiss_r, heads_r, rels_r, tails_r, dateid_r,
               ent_w, rel_p, tim_full)
